# baseline (speedup 1.0000x reference)
import functools

import jax
import jax.numpy as jnp
from jax import lax
from jax.experimental import pallas as pl
from jax.experimental.pallas import tpu as pltpu
from jax.experimental.pallas import tpu_sc as plsc

NUM_CORES = 2
NUM_SUBCORES = 16
LANES = 16
NW = NUM_CORES * NUM_SUBCORES

D = 64
QD = D // LANES
B = 16384
BPW = B // NW
NG = BPW // LANES
NR = 100
NUM_ENT = 1000000


def _rsqrt_v(x):
    i = lax.bitcast_convert_type(x, jnp.int32)
    i = jnp.int32(0x5F3759DF) - (i >> 1)
    y = lax.bitcast_convert_type(i, jnp.float32)
    half_x = 0.5 * x
    y = y * (1.5 - half_x * y * y)
    y = y * (1.5 - half_x * y * y)
    y = y * (1.5 - half_x * y * y)
    return y


_SHUFFLE_DN = lax.GatherDimensionNumbers(
    offset_dims=(), collapsed_slice_dims=(0,), start_index_map=(0,))


def _shuffle(v, idx):
    return lax.gather(v, idx[:, None], _SHUFFLE_DN, (1,),
                      mode=lax.GatherScatterMode.PROMISE_IN_BOUNDS)


def _hsum(v, lane):
    for sh in (8, 4, 2, 1):
        v = v + _shuffle(v, lane ^ sh)
    return v


def _sc_body(hidx_hbm, ridx_hbm, tidx_hbm, ent_hbm, rel_hbm, out_hbm,
             hidx_v, ridx_v, tidx_v, hb, tb, rel_v, scores, sem, semb):
    wid = lax.axis_index("s") * NUM_CORES + lax.axis_index("c")
    base = wid * BPW

    pltpu.sync_copy(hidx_hbm.at[wid], hidx_v)
    pltpu.sync_copy(ridx_hbm.at[wid], ridx_v)
    pltpu.sync_copy(tidx_hbm.at[wid], tidx_v)
    pltpu.sync_copy(rel_hbm, rel_v)

    lane = lax.iota(jnp.int32, LANES)

    sems = (sem, semb)

    def fire_quarter(q):
        def fire_group(g, carry, q=q):
            gg = q * 8 + g
            col = (gg & 7) * LANES
            hid = hidx_v[q, pl.ds(g * LANES, LANES)]
            tid = tidx_v[q, pl.ds(g * LANES, LANES)]
            for j in range(LANES):
                i = g * LANES + j
                he = hid[j]
                te = tid[j]
                pltpu.async_copy(ent_hbm.at[he >> 3, he & 7],
                                 hb.at[q & 1, i >> 3, i & 7], sems[q & 1])
                pltpu.async_copy(ent_hbm.at[te >> 3, te & 7],
                                 tb.at[q & 1, i >> 3, i & 7], sems[q & 1])
            return carry
        lax.fori_loop(0, 8, fire_group, 0)

    fire_quarter(0)
    for q in range(4):
        if q + 1 < 4:
            fire_quarter(q + 1)
        pltpu.make_async_copy(ent_hbm.at[pl.ds(0, 16)], hb.at[q & 1],
                              sems[q & 1]).wait()
        pltpu.make_async_copy(ent_hbm.at[pl.ds(0, 16)], tb.at[q & 1],
                              sems[q & 1]).wait()

        def group(g, carry, q=q):
            rid = ridx_v[q, pl.ds(g * LANES, LANES)]
            nh = jnp.zeros((LANES,), jnp.float32)
            nt = jnp.zeros((LANES,), jnp.float32)
            for j in range(LANES):
                i = g * LANES + j
                ph = pt = None
                for qq in range(QD):
                    hv = hb[q & 1, i >> 3, i & 7, pl.ds(qq * LANES, LANES)]
                    tv = tb[q & 1, i >> 3, i & 7, pl.ds(qq * LANES, LANES)]
                    ph = hv * hv if ph is None else ph + hv * hv
                    pt = tv * tv if pt is None else pt + tv * tv
                m = lane == j
                nh = jnp.where(m, _hsum(ph, lane), nh)
                nt = jnp.where(m, _hsum(pt, lane), nt)
            rsh_vec = _rsqrt_v(jnp.maximum(nh, 1e-30))
            rst_vec = _rsqrt_v(jnp.maximum(nt, 1e-30))
            s2v = jnp.zeros((LANES,), jnp.float32)
            for j in range(LANES):
                i = g * LANES + j
                bj = jnp.full((LANES,), j, jnp.int32)
                a = _shuffle(rsh_vec, bj)
                b = _shuffle(rst_vec, bj)
                rrow = rid[j]
                ps = None
                for qq in range(QD):
                    hv = hb[q & 1, i >> 3, i & 7, pl.ds(qq * LANES, LANES)]
                    tv = tb[q & 1, i >> 3, i & 7, pl.ds(qq * LANES, LANES)]
                    rv = rel_v[rrow, pl.ds(qq * LANES, LANES)]
                    dv = hv * a + rv - tv * b
                    ps = dv * dv if ps is None else ps + dv * dv
                s2v = jnp.where(lane == j, _hsum(ps, lane), s2v)
            scores[pl.ds(q * 128 + g * LANES, LANES)] = s2v * _rsqrt_v(
                jnp.maximum(s2v, 1e-30))
            return carry

        lax.fori_loop(0, 8, group, 0)

    pltpu.sync_copy(scores, out_hbm.at[pl.ds(base, BPW)])


@jax.jit
def _transe_sc(heads_r, relations_r, tails_r, entity_emb, relation_emb):
    mesh = plsc.VectorSubcoreMesh(
        core_axis_name="c", subcore_axis_name="s",
        num_cores=NUM_CORES, num_subcores=NUM_SUBCORES)
    return pl.kernel(
        _sc_body,
        out_type=jax.ShapeDtypeStruct((B,), jnp.float32),
        mesh=mesh,
        compiler_params=pltpu.CompilerParams(use_tc_tiling_on_sc=True),
        scratch_types=[
            pltpu.VMEM((4, 128), jnp.int32),
            pltpu.VMEM((4, 128), jnp.int32),
            pltpu.VMEM((4, 128), jnp.int32),
            pltpu.VMEM((2, 16, 8, D), jnp.float32),
            pltpu.VMEM((2, 16, 8, D), jnp.float32),
            pltpu.VMEM((NR, D), jnp.float32),
            pltpu.VMEM((BPW,), jnp.float32),
            pltpu.SemaphoreType.DMA,
            pltpu.SemaphoreType.DMA,
        ],
    )(heads_r, relations_r, tails_r, entity_emb, relation_emb)


def kernel(heads, relations, tails, entity_emb, relation_emb):
    heads_r = heads.reshape(NW, 4, 128)
    relations_r = relations.reshape(NW, 4, 128)
    tails_r = tails.reshape(NW, 4, 128)
    entity_r = entity_emb.reshape(NUM_ENT // 8, 8, D)
    return _transe_sc(heads_r, relations_r, tails_r, entity_r, relation_emb)


# speedup vs baseline: 1.0076x; 1.0076x over previous
import functools

import jax
import jax.numpy as jnp
from jax import lax
from jax.experimental import pallas as pl
from jax.experimental.pallas import tpu as pltpu
from jax.experimental.pallas import tpu_sc as plsc

NUM_CORES = 2
NUM_SUBCORES = 16
LANES = 16
NW = NUM_CORES * NUM_SUBCORES

D = 64
QD = D // LANES
B = 16384
BPW = B // NW
NG = BPW // LANES
NR = 100
NUM_ENT = 1000000


def _rsqrt_v(x):
    i = lax.bitcast_convert_type(x, jnp.int32)
    i = jnp.int32(0x5F3759DF) - (i >> 1)
    y = lax.bitcast_convert_type(i, jnp.float32)
    half_x = 0.5 * x
    y = y * (1.5 - half_x * y * y)
    y = y * (1.5 - half_x * y * y)
    y = y * (1.5 - half_x * y * y)
    return y


_SHUFFLE_DN = lax.GatherDimensionNumbers(
    offset_dims=(), collapsed_slice_dims=(0,), start_index_map=(0,))


def _shuffle(v, idx):
    return lax.gather(v, idx[:, None], _SHUFFLE_DN, (1,),
                      mode=lax.GatherScatterMode.PROMISE_IN_BOUNDS)


def _hsum(v, lane):
    for sh in (8, 4, 2, 1):
        v = v + _shuffle(v, lane ^ sh)
    return v


def _sc_body(hidx_hbm, ridx_hbm, tidx_hbm, ent_hbm, rel_hbm, out_hbm,
             hidx_v, ridx_v, tidx_v, hb, tb, rel_v, scores, sem):
    wid = lax.axis_index("s") * NUM_CORES + lax.axis_index("c")
    base = wid * BPW

    pltpu.sync_copy(hidx_hbm.at[wid], hidx_v)
    pltpu.sync_copy(ridx_hbm.at[wid], ridx_v)
    pltpu.sync_copy(tidx_hbm.at[wid], tidx_v)
    pltpu.sync_copy(rel_hbm, rel_v)

    lane = lax.iota(jnp.int32, LANES)

    for half in range(2):
        goff = half * (NG // 2)

        def fire_group(g, carry, goff=goff):
            gg = g + goff
            row = gg >> 3
            col = (gg & 7) * LANES
            hid = hidx_v[row, pl.ds(col, LANES)]
            tid = tidx_v[row, pl.ds(col, LANES)]
            for j in range(LANES):
                i = g * LANES + j
                he = hid[j]
                te = tid[j]
                pltpu.async_copy(ent_hbm.at[he >> 3, he & 7], hb.at[i >> 3, i & 7], sem)
                pltpu.async_copy(ent_hbm.at[te >> 3, te & 7], tb.at[i >> 3, i & 7], sem)
            return carry

        lax.fori_loop(0, NG // 2, fire_group, 0)
        pltpu.make_async_copy(ent_hbm.at[pl.ds(0, (BPW // 2) // 8)], hb, sem).wait()
        pltpu.make_async_copy(ent_hbm.at[pl.ds(0, (BPW // 2) // 8)], tb, sem).wait()

        def group(g, carry, goff=goff):
            gg = g + goff
            row = gg >> 3
            col = (gg & 7) * LANES
            rid = ridx_v[row, pl.ds(col, LANES)]
            nh = jnp.zeros((LANES,), jnp.float32)
            nt = jnp.zeros((LANES,), jnp.float32)
            for j in range(LANES):
                i = g * LANES + j
                ph = pt = None
                for q in range(QD):
                    hv = hb[i >> 3, i & 7, pl.ds(q * LANES, LANES)]
                    tv = tb[i >> 3, i & 7, pl.ds(q * LANES, LANES)]
                    ph = hv * hv if ph is None else ph + hv * hv
                    pt = tv * tv if pt is None else pt + tv * tv
                m = lane == j
                nh = jnp.where(m, _hsum(ph, lane), nh)
                nt = jnp.where(m, _hsum(pt, lane), nt)
            rsh_vec = _rsqrt_v(jnp.maximum(nh, 1e-30))
            rst_vec = _rsqrt_v(jnp.maximum(nt, 1e-30))
            s2v = jnp.zeros((LANES,), jnp.float32)
            for j in range(LANES):
                i = g * LANES + j
                bj = jnp.full((LANES,), j, jnp.int32)
                a = _shuffle(rsh_vec, bj)
                b = _shuffle(rst_vec, bj)
                rrow = rid[j]
                ps = None
                for q in range(QD):
                    hv = hb[i >> 3, i & 7, pl.ds(q * LANES, LANES)]
                    tv = tb[i >> 3, i & 7, pl.ds(q * LANES, LANES)]
                    rv = rel_v[rrow, pl.ds(q * LANES, LANES)]
                    dv = hv * a + rv - tv * b
                    ps = dv * dv if ps is None else ps + dv * dv
                s2v = jnp.where(lane == j, _hsum(ps, lane), s2v)
            scores[pl.ds(gg * LANES, LANES)] = s2v * _rsqrt_v(
                jnp.maximum(s2v, 1e-30))
            return carry

        lax.fori_loop(0, NG // 2, group, 0)

    pltpu.sync_copy(scores, out_hbm.at[pl.ds(base, BPW)])


@jax.jit
def _transe_sc(heads_r, relations_r, tails_r, entity_emb, relation_emb):
    mesh = plsc.VectorSubcoreMesh(
        core_axis_name="c", subcore_axis_name="s",
        num_cores=NUM_CORES, num_subcores=NUM_SUBCORES)
    return pl.kernel(
        _sc_body,
        out_type=jax.ShapeDtypeStruct((B,), jnp.float32),
        mesh=mesh,
        compiler_params=pltpu.CompilerParams(use_tc_tiling_on_sc=True),
        scratch_types=[
            pltpu.VMEM((4, 128), jnp.int32),
            pltpu.VMEM((4, 128), jnp.int32),
            pltpu.VMEM((4, 128), jnp.int32),
            pltpu.VMEM((BPW // 16, 8, D), jnp.float32),
            pltpu.VMEM((BPW // 16, 8, D), jnp.float32),
            pltpu.VMEM((NR, D), jnp.float32),
            pltpu.VMEM((BPW,), jnp.float32),
            pltpu.SemaphoreType.DMA,
        ],
    )(heads_r, relations_r, tails_r, entity_emb, relation_emb)


def kernel(heads, relations, tails, entity_emb, relation_emb):
    heads_r = heads.reshape(NW, 4, 128)
    relations_r = relations.reshape(NW, 4, 128)
    tails_r = tails.reshape(NW, 4, 128)
    entity_r = entity_emb.reshape(NUM_ENT // 8, 8, D)
    return _transe_sc(heads_r, relations_r, tails_r, entity_r, relation_emb)
